# baseline (speedup 1.0000x reference)
import functools

import jax
import jax.numpy as jnp
from jax import lax
from jax.experimental import pallas as pl
from jax.experimental.pallas import tpu as pltpu
from jax.experimental.pallas import tpu_sc as plsc

N = 10000
DEG = 32
E = N * DEG
HID = 128
HEADS = 4
DK = HID // HEADS

B_A = 400
B_B = 400

NSLC = 5
ESL = E // NSLC
NW = 32
PER_W = ESL // NW
CH = 80
NCH = PER_W // CH


def _attn_body(f_ref, ex_ref, wq_ref, bq_ref, wk_ref, bk_ref, wv_ref, bv_ref,
               wo_ref, bo_ref, sel_ref, selt_ref, fh_ref):
    b = f_ref.shape[0]
    fb = f_ref[...]
    ex = ex_ref[...]
    exb = ex.astype(jnp.bfloat16)
    q = jnp.dot(fb.astype(jnp.bfloat16), wq_ref[...],
                preferred_element_type=jnp.float32) + bq_ref[...]
    k = jnp.dot(exb, wk_ref[...], preferred_element_type=jnp.float32) + bk_ref[...]
    v = jnp.dot(exb, wv_ref[...], preferred_element_type=jnp.float32) + bv_ref[...]
    k3 = k.reshape(b, DEG, HID)
    qk = (k3 * q[:, None, :]).reshape(b * DEG, HID)
    s = jnp.dot(qk, sel_ref[...], preferred_element_type=jnp.float32) * (DK ** -0.5)
    e = jnp.exp(s)
    ef = jnp.dot(e, selt_ref[...], preferred_element_type=jnp.float32)
    ef3 = ef.reshape(b, DEG, HID)
    num = jnp.sum(ef3 * v.reshape(b, DEG, HID), axis=1)
    den = jnp.sum(ef3, axis=1)
    x = num / den
    attn = jnp.dot(x.astype(jnp.bfloat16), wo_ref[...],
                   preferred_element_type=jnp.float32) + bo_ref[...]
    fh_ref[...] = attn + fb


def _pairswap(x):
    up = pltpu.roll(x, x.shape[0] - 1, 0)
    dn = pltpu.roll(x, 1, 0)
    par = lax.broadcasted_iota(jnp.int32, x.shape, 0) % 2
    return jnp.where(par == 0, up, dn)


def _edge_body(ex_ref, g_ref, fh_ref, f_ref, wmp_ref, bmp_ref,
               w1_ref, w2_ref, w3_ref, bl_ref, out_ref):
    b = fh_ref.shape[0]
    ex = ex_ref[...]
    g = g_ref[...]
    rev = _pairswap(ex)
    t = jnp.dot((g - rev).astype(jnp.bfloat16), wmp_ref[...],
                preferred_element_type=jnp.float32) + bmp_ref[...]
    h = jnp.maximum(ex + t, 0.0)
    ms = jnp.sum(h.reshape(b, DEG, HID), axis=1)
    out = (jnp.dot(ms.astype(jnp.bfloat16), w1_ref[...],
                   preferred_element_type=jnp.float32)
           + jnp.dot(fh_ref[...].astype(jnp.bfloat16), w2_ref[...],
                     preferred_element_type=jnp.float32)
           + jnp.dot(f_ref[...].astype(jnp.bfloat16), w3_ref[...],
                     preferred_element_type=jnp.float32)
           + bl_ref[...])
    out_ref[...] = out


def _full(shape):
    return pl.BlockSpec(shape, lambda i: (0, 0))


@functools.lru_cache(maxsize=1)
def _sc_gather_fn():
    mesh = plsc.VectorSubcoreMesh(core_axis_name="c", subcore_axis_name="s")

    @functools.partial(
        pl.kernel,
        mesh=mesh,
        out_type=jax.ShapeDtypeStruct((ESL, HID), jnp.float32),
        scratch_types=[
            pltpu.VMEM((NCH, CH), jnp.int32),
            pltpu.VMEM((CH, HID), jnp.float32),
            pltpu.VMEM((CH, HID), jnp.float32),
            pltpu.SemaphoreType.DMA,
            pltpu.SemaphoreType.DMA,
        ],
    )
    def _sc_gather(table_hbm, idx_hbm, out_hbm, idx_v, buf0, buf1, sem0, sem1):
        w = lax.axis_index("s") * 2 + lax.axis_index("c")
        pltpu.sync_copy(idx_hbm.at[w], idx_v)
        base = w * PER_W

        pltpu.async_copy(table_hbm.at[idx_v.at[0]], buf0, sem0)

        def body(i, carry):
            j = 2 * i
            pltpu.async_copy(table_hbm.at[idx_v.at[j + 1]], buf1, sem1)
            pltpu.make_async_copy(table_hbm.at[idx_v.at[j]], buf0, sem0).wait()
            pltpu.sync_copy(buf0, out_hbm.at[pl.ds(base + j * CH, CH)])
            pltpu.async_copy(table_hbm.at[idx_v.at[j + 2]], buf0, sem0)
            pltpu.make_async_copy(table_hbm.at[idx_v.at[j + 1]], buf1, sem1).wait()
            pltpu.sync_copy(buf1, out_hbm.at[pl.ds(base + (j + 1) * CH, CH)])
            return carry

        lax.fori_loop(0, (NCH - 1) // 2, body, 0)
        pltpu.make_async_copy(table_hbm.at[idx_v.at[NCH - 1]], buf0, sem0).wait()
        pltpu.sync_copy(buf0, out_hbm.at[pl.ds(base + (NCH - 1) * CH, CH)])

    return _sc_gather


def kernel(f, edge_src, edge_x, Wq, bq, Wk, bk, Wv, bv, Wo, bo,
           W_mp0, b_mp0, W_last, b_last):
    bf = jnp.bfloat16
    wqT, wkT, wvT, woT, wmpT = (Wq.T.astype(bf), Wk.T.astype(bf),
                                Wv.T.astype(bf), Wo.T.astype(bf),
                                W_mp0.T.astype(bf))
    wlT = W_last.T.astype(bf)
    w1, w2, w3 = wlT[:HID], wlT[HID:2 * HID], wlT[2 * HID:]
    sel = (jnp.arange(HID)[:, None] // DK
           == jnp.arange(HEADS)[None, :]).astype(jnp.float32)
    selt = sel.T
    bq2, bk2, bv2, bo2 = bq[None], bk[None], bv[None], bo[None]
    bmp2, bl2 = b_mp0[None], b_last[None]

    fh = pl.pallas_call(
        _attn_body,
        grid=(N // B_A,),
        in_specs=[
            pl.BlockSpec((B_A, HID), lambda i: (i, 0)),
            pl.BlockSpec((B_A * DEG, HID), lambda i: (i, 0)),
            _full((HID, HID)), _full((1, HID)),
            _full((HID, HID)), _full((1, HID)),
            _full((HID, HID)), _full((1, HID)),
            _full((HID, HID)), _full((1, HID)),
            _full((HID, HEADS)), _full((HEADS, HID)),
        ],
        out_specs=pl.BlockSpec((B_A, HID), lambda i: (i, 0)),
        out_shape=jax.ShapeDtypeStruct((N, HID), jnp.float32),
        compiler_params=pltpu.CompilerParams(
            vmem_limit_bytes=100 * 1024 * 1024),
    )(f, edge_x, wqT, bq2, wkT, bk2, wvT, bv2, woT, bo2, sel, selt)

    idx4 = edge_src.reshape(NSLC, NW, NCH, CH)
    gather = _sc_gather_fn()
    gs = [gather(fh, idx4[s]) for s in range(NSLC)]

    nsl = N // NSLC
    nblk = nsl // B_B
    outs = []
    for s in range(NSLC):
        out_s = pl.pallas_call(
            _edge_body,
            grid=(nblk,),
            in_specs=[
                pl.BlockSpec((B_B * DEG, HID),
                             lambda i, s=s: (i + s * nblk, 0)),
                pl.BlockSpec((B_B * DEG, HID), lambda i: (i, 0)),
                pl.BlockSpec((B_B, HID), lambda i, s=s: (i + s * nblk, 0)),
                pl.BlockSpec((B_B, HID), lambda i, s=s: (i + s * nblk, 0)),
                _full((HID, HID)), _full((1, HID)),
                _full((HID, HID)), _full((HID, HID)), _full((HID, HID)),
                _full((1, HID)),
            ],
            out_specs=pl.BlockSpec((B_B, HID), lambda i: (i, 0)),
            out_shape=jax.ShapeDtypeStruct((nsl, HID), jnp.float32),
            compiler_params=pltpu.CompilerParams(
                vmem_limit_bytes=100 * 1024 * 1024),
        )(edge_x, gs[s], fh, f, wmpT, bmp2, w1, w2, w3, bl2)
        outs.append(out_s)
    return jnp.concatenate(outs, axis=0)


# speedup vs baseline: 1.0126x; 1.0126x over previous
import functools

import jax
import jax.numpy as jnp
from jax import lax
from jax.experimental import pallas as pl
from jax.experimental.pallas import tpu as pltpu
from jax.experimental.pallas import tpu_sc as plsc

N = 10000
DEG = 32
E = N * DEG
HID = 128
HEADS = 4
DK = HID // HEADS

B_A = 400
B_B = 400

NSLC = 1
ESL = E // NSLC
NW = 32
PER_W = ESL // NW
CH = 80
NCH = PER_W // CH


def _attn_body(f_ref, ex_ref, wq_ref, bq_ref, wk_ref, bk_ref, wv_ref, bv_ref,
               wo_ref, bo_ref, sel_ref, selt_ref, fh_ref, exb_ref):
    b = f_ref.shape[0]
    fb = f_ref[...]
    ex = ex_ref[...]
    exb = ex.astype(jnp.bfloat16)
    exb_ref[...] = exb
    q = jnp.dot(fb.astype(jnp.bfloat16), wq_ref[...],
                preferred_element_type=jnp.float32) + bq_ref[...]
    k = jnp.dot(exb, wk_ref[...], preferred_element_type=jnp.float32) + bk_ref[...]
    v = jnp.dot(exb, wv_ref[...], preferred_element_type=jnp.float32) + bv_ref[...]
    k3 = k.reshape(b, DEG, HID)
    qk = (k3 * q[:, None, :]).reshape(b * DEG, HID)
    s = jnp.dot(qk, sel_ref[...], preferred_element_type=jnp.float32) * (DK ** -0.5)
    e = jnp.exp(s)
    ef = jnp.dot(e, selt_ref[...], preferred_element_type=jnp.float32)
    ef3 = ef.reshape(b, DEG, HID)
    num = jnp.sum(ef3 * v.reshape(b, DEG, HID), axis=1)
    den = jnp.sum(ef3, axis=1)
    x = num / den
    attn = jnp.dot(x.astype(jnp.bfloat16), wo_ref[...],
                   preferred_element_type=jnp.float32) + bo_ref[...]
    fh_ref[...] = attn + fb


def _pairswap(x):
    up = pltpu.roll(x, x.shape[0] - 1, 0)
    dn = pltpu.roll(x, 1, 0)
    par = lax.broadcasted_iota(jnp.int32, x.shape, 0) % 2
    return jnp.where(par == 0, up, dn)


def _edge_body(ex_ref, g_ref, fh_ref, f_ref, wmp_ref, bmp_ref,
               w1_ref, w2_ref, w3_ref, bl_ref, out_ref):
    b = fh_ref.shape[0]
    exb = ex_ref[...]
    g = g_ref[...]
    rev = _pairswap(exb)
    t = jnp.dot(g.astype(jnp.bfloat16) - rev, wmp_ref[...],
                preferred_element_type=jnp.float32) + bmp_ref[...]
    h = jnp.maximum(exb.astype(jnp.float32) + t, 0.0)
    ms = jnp.sum(h.reshape(b, DEG, HID), axis=1)
    out = (jnp.dot(ms.astype(jnp.bfloat16), w1_ref[...],
                   preferred_element_type=jnp.float32)
           + jnp.dot(fh_ref[...].astype(jnp.bfloat16), w2_ref[...],
                     preferred_element_type=jnp.float32)
           + jnp.dot(f_ref[...].astype(jnp.bfloat16), w3_ref[...],
                     preferred_element_type=jnp.float32)
           + bl_ref[...])
    out_ref[...] = out


def _full(shape):
    return pl.BlockSpec(shape, lambda i: (0, 0))


@functools.lru_cache(maxsize=1)
def _sc_gather_fn():
    mesh = plsc.VectorSubcoreMesh(core_axis_name="c", subcore_axis_name="s")

    @functools.partial(
        pl.kernel,
        mesh=mesh,
        out_type=jax.ShapeDtypeStruct((ESL, HID), jnp.float32),
        scratch_types=[
            pltpu.VMEM((NCH, CH), jnp.int32),
            pltpu.VMEM((CH, HID), jnp.float32),
            pltpu.VMEM((CH, HID), jnp.float32),
            pltpu.SemaphoreType.DMA,
            pltpu.SemaphoreType.DMA,
        ],
    )
    def _sc_gather(table_hbm, idx_hbm, out_hbm, idx_v, buf0, buf1, sem0, sem1):
        w = lax.axis_index("s") * 2 + lax.axis_index("c")
        pltpu.sync_copy(idx_hbm.at[w], idx_v)
        base = w * PER_W

        pltpu.async_copy(table_hbm.at[idx_v.at[0]], buf0, sem0)

        def body(i, carry):
            j = 2 * i
            pltpu.async_copy(table_hbm.at[idx_v.at[j + 1]], buf1, sem1)
            pltpu.make_async_copy(table_hbm.at[idx_v.at[j]], buf0, sem0).wait()
            pltpu.sync_copy(buf0, out_hbm.at[pl.ds(base + j * CH, CH)])
            pltpu.async_copy(table_hbm.at[idx_v.at[j + 2]], buf0, sem0)
            pltpu.make_async_copy(table_hbm.at[idx_v.at[j + 1]], buf1, sem1).wait()
            pltpu.sync_copy(buf1, out_hbm.at[pl.ds(base + (j + 1) * CH, CH)])
            return carry

        lax.fori_loop(0, (NCH - 1) // 2, body, 0)
        pltpu.make_async_copy(table_hbm.at[idx_v.at[NCH - 1]], buf0, sem0).wait()
        pltpu.sync_copy(buf0, out_hbm.at[pl.ds(base + (NCH - 1) * CH, CH)])

    return _sc_gather


def kernel(f, edge_src, edge_x, Wq, bq, Wk, bk, Wv, bv, Wo, bo,
           W_mp0, b_mp0, W_last, b_last):
    bf = jnp.bfloat16
    wqT, wkT, wvT, woT, wmpT = (Wq.T.astype(bf), Wk.T.astype(bf),
                                Wv.T.astype(bf), Wo.T.astype(bf),
                                W_mp0.T.astype(bf))
    wlT = W_last.T.astype(bf)
    w1, w2, w3 = wlT[:HID], wlT[HID:2 * HID], wlT[2 * HID:]
    sel = (jnp.arange(HID)[:, None] // DK
           == jnp.arange(HEADS)[None, :]).astype(jnp.float32)
    selt = sel.T
    bq2, bk2, bv2, bo2 = bq[None], bk[None], bv[None], bo[None]
    bmp2, bl2 = b_mp0[None], b_last[None]

    fh = pl.pallas_call(
        _attn_body,
        grid=(N // B_A,),
        in_specs=[
            pl.BlockSpec((B_A, HID), lambda i: (i, 0)),
            pl.BlockSpec((B_A * DEG, HID), lambda i: (i, 0)),
            _full((HID, HID)), _full((1, HID)),
            _full((HID, HID)), _full((1, HID)),
            _full((HID, HID)), _full((1, HID)),
            _full((HID, HID)), _full((1, HID)),
            _full((HID, HEADS)), _full((HEADS, HID)),
        ],
        out_specs=[pl.BlockSpec((B_A, HID), lambda i: (i, 0)),
                   pl.BlockSpec((B_A * DEG, HID), lambda i: (i, 0))],
        out_shape=[jax.ShapeDtypeStruct((N, HID), jnp.float32),
                   jax.ShapeDtypeStruct((E, HID), jnp.bfloat16)],
        compiler_params=pltpu.CompilerParams(
            vmem_limit_bytes=100 * 1024 * 1024),
    )(f, edge_x, wqT, bq2, wkT, bk2, wvT, bv2, woT, bo2, sel, selt)
    fh, exbf = fh

    idx3 = edge_src.reshape(NW, NCH, CH)
    g = _sc_gather_fn()(fh, idx3)

    out = pl.pallas_call(
        _edge_body,
        grid=(N // B_B,),
        in_specs=[
            pl.BlockSpec((B_B * DEG, HID), lambda i: (i, 0)),
            pl.BlockSpec((B_B * DEG, HID), lambda i: (i, 0)),
            pl.BlockSpec((B_B, HID), lambda i: (i, 0)),
            pl.BlockSpec((B_B, HID), lambda i: (i, 0)),
            _full((HID, HID)), _full((1, HID)),
            _full((HID, HID)), _full((HID, HID)), _full((HID, HID)),
            _full((1, HID)),
        ],
        out_specs=pl.BlockSpec((B_B, HID), lambda i: (i, 0)),
        out_shape=jax.ShapeDtypeStruct((N, HID), jnp.float32),
        compiler_params=pltpu.CompilerParams(
            vmem_limit_bytes=100 * 1024 * 1024),
    )(exbf, g, fh, f, wmpT, bmp2, w1, w2, w3, bl2)
    return out


# speedup vs baseline: 1.1941x; 1.1793x over previous
import functools

import jax
import jax.numpy as jnp
from jax import lax
from jax.experimental import pallas as pl
from jax.experimental.pallas import tpu as pltpu
from jax.experimental.pallas import tpu_sc as plsc

N = 10000
DEG = 32
E = N * DEG
HID = 128
HEADS = 4
DK = HID // HEADS

B_A = 400
B_B = 400

NSLC = 1
ESL = E // NSLC
NW = 32
PER_W = ESL // NW
CH = 80
NCH = PER_W // CH


def _attn_body(f_ref, ex_ref, wq_ref, bq_ref, wk_ref, bk_ref, wv_ref, bv_ref,
               wo_ref, bo_ref, sel_ref, selt_ref, fh_ref, exb_ref):
    b = f_ref.shape[0]
    fb = f_ref[...]
    ex = ex_ref[...]
    exb = ex.astype(jnp.bfloat16)
    exb_ref[...] = exb
    q = jnp.dot(fb.astype(jnp.bfloat16), wq_ref[...],
                preferred_element_type=jnp.float32) + bq_ref[...]
    k = jnp.dot(exb, wk_ref[...], preferred_element_type=jnp.float32) + bk_ref[...]
    v = jnp.dot(exb, wv_ref[...], preferred_element_type=jnp.float32) + bv_ref[...]
    k3 = k.reshape(b, DEG, HID)
    qk = (k3 * q[:, None, :]).reshape(b * DEG, HID)
    s = jnp.dot(qk, sel_ref[...], preferred_element_type=jnp.float32) * (DK ** -0.5)
    e = jnp.exp(s)
    ef = jnp.dot(e, selt_ref[...], preferred_element_type=jnp.float32)
    ef3 = ef.reshape(b, DEG, HID)
    num = jnp.sum(ef3 * v.reshape(b, DEG, HID), axis=1)
    den = jnp.sum(ef3, axis=1)
    x = num / den
    attn = jnp.dot(x.astype(jnp.bfloat16), wo_ref[...],
                   preferred_element_type=jnp.float32) + bo_ref[...]
    fh_ref[...] = attn + fb


def _pairswap(x):
    up = pltpu.roll(x, x.shape[0] - 1, 0)
    dn = pltpu.roll(x, 1, 0)
    par = lax.broadcasted_iota(jnp.int32, x.shape, 0) % 2
    return jnp.where(par == 0, up, dn)


def _edge_body(ex_ref, g_ref, fh_ref, f_ref, wmp_ref, bmp_ref,
               w1_ref, w2_ref, w3_ref, bl_ref, out_ref):
    b = fh_ref.shape[0]
    exb = ex_ref[...]
    g = g_ref[...]
    rev = _pairswap(exb)
    t = jnp.dot(g.astype(jnp.bfloat16) - rev, wmp_ref[...],
                preferred_element_type=jnp.float32) + bmp_ref[...]
    h = jnp.maximum(exb.astype(jnp.float32) + t, 0.0)
    ms = jnp.sum(h.reshape(b, DEG, HID), axis=1)
    out = (jnp.dot(ms.astype(jnp.bfloat16), w1_ref[...],
                   preferred_element_type=jnp.float32)
           + jnp.dot(fh_ref[...].astype(jnp.bfloat16), w2_ref[...],
                     preferred_element_type=jnp.float32)
           + jnp.dot(f_ref[...].astype(jnp.bfloat16), w3_ref[...],
                     preferred_element_type=jnp.float32)
           + bl_ref[...])
    out_ref[...] = out


def _full(shape):
    return pl.BlockSpec(shape, lambda i: (0, 0))


@functools.lru_cache(maxsize=1)
def _sc_gather_fn():
    mesh = plsc.VectorSubcoreMesh(core_axis_name="c", subcore_axis_name="s")

    @functools.partial(
        pl.kernel,
        mesh=mesh,
        out_type=jax.ShapeDtypeStruct((ESL, HID), jnp.float32),
        scratch_types=[
            pltpu.VMEM((NCH, CH), jnp.int32),
            pltpu.VMEM((CH, HID), jnp.float32),
            pltpu.VMEM((CH, HID), jnp.float32),
            pltpu.VMEM_SHARED((N, HID), jnp.float32),
            pltpu.SemaphoreType.DMA,
            pltpu.SemaphoreType.DMA,
        ],
    )
    def _sc_gather(table_hbm, idx_hbm, out_hbm, idx_v, buf0, buf1, sp,
                   sem0, sem1):
        sid = lax.axis_index("s")
        w = sid * 2 + lax.axis_index("c")
        @pl.when(sid < 10)
        def _load():
            pltpu.sync_copy(table_hbm.at[pl.ds(sid * 1000, 1000)],
                            sp.at[pl.ds(sid * 1000, 1000)])
        plsc.subcore_barrier()
        pltpu.sync_copy(idx_hbm.at[w], idx_v)
        base = w * PER_W
        table_hbm = sp

        pltpu.async_copy(table_hbm.at[idx_v.at[0]], buf0, sem0)

        def body(i, carry):
            j = 2 * i
            pltpu.async_copy(table_hbm.at[idx_v.at[j + 1]], buf1, sem1)
            pltpu.make_async_copy(table_hbm.at[idx_v.at[j]], buf0, sem0).wait()
            pltpu.sync_copy(buf0, out_hbm.at[pl.ds(base + j * CH, CH)])
            pltpu.async_copy(table_hbm.at[idx_v.at[j + 2]], buf0, sem0)
            pltpu.make_async_copy(table_hbm.at[idx_v.at[j + 1]], buf1, sem1).wait()
            pltpu.sync_copy(buf1, out_hbm.at[pl.ds(base + (j + 1) * CH, CH)])
            return carry

        lax.fori_loop(0, (NCH - 1) // 2, body, 0)
        pltpu.make_async_copy(table_hbm.at[idx_v.at[NCH - 1]], buf0, sem0).wait()
        pltpu.sync_copy(buf0, out_hbm.at[pl.ds(base + (NCH - 1) * CH, CH)])

    return _sc_gather


def kernel(f, edge_src, edge_x, Wq, bq, Wk, bk, Wv, bv, Wo, bo,
           W_mp0, b_mp0, W_last, b_last):
    bf = jnp.bfloat16
    wqT, wkT, wvT, woT, wmpT = (Wq.T.astype(bf), Wk.T.astype(bf),
                                Wv.T.astype(bf), Wo.T.astype(bf),
                                W_mp0.T.astype(bf))
    wlT = W_last.T.astype(bf)
    w1, w2, w3 = wlT[:HID], wlT[HID:2 * HID], wlT[2 * HID:]
    sel = (jnp.arange(HID)[:, None] // DK
           == jnp.arange(HEADS)[None, :]).astype(jnp.float32)
    selt = sel.T
    bq2, bk2, bv2, bo2 = bq[None], bk[None], bv[None], bo[None]
    bmp2, bl2 = b_mp0[None], b_last[None]

    fh = pl.pallas_call(
        _attn_body,
        grid=(N // B_A,),
        in_specs=[
            pl.BlockSpec((B_A, HID), lambda i: (i, 0)),
            pl.BlockSpec((B_A * DEG, HID), lambda i: (i, 0)),
            _full((HID, HID)), _full((1, HID)),
            _full((HID, HID)), _full((1, HID)),
            _full((HID, HID)), _full((1, HID)),
            _full((HID, HID)), _full((1, HID)),
            _full((HID, HEADS)), _full((HEADS, HID)),
        ],
        out_specs=[pl.BlockSpec((B_A, HID), lambda i: (i, 0)),
                   pl.BlockSpec((B_A * DEG, HID), lambda i: (i, 0))],
        out_shape=[jax.ShapeDtypeStruct((N, HID), jnp.float32),
                   jax.ShapeDtypeStruct((E, HID), jnp.bfloat16)],
        compiler_params=pltpu.CompilerParams(
            vmem_limit_bytes=100 * 1024 * 1024),
    )(f, edge_x, wqT, bq2, wkT, bk2, wvT, bv2, woT, bo2, sel, selt)
    fh, exbf = fh

    idx3 = edge_src.reshape(NW, NCH, CH)
    g = _sc_gather_fn()(fh, idx3)

    out = pl.pallas_call(
        _edge_body,
        grid=(N // B_B,),
        in_specs=[
            pl.BlockSpec((B_B * DEG, HID), lambda i: (i, 0)),
            pl.BlockSpec((B_B * DEG, HID), lambda i: (i, 0)),
            pl.BlockSpec((B_B, HID), lambda i: (i, 0)),
            pl.BlockSpec((B_B, HID), lambda i: (i, 0)),
            _full((HID, HID)), _full((1, HID)),
            _full((HID, HID)), _full((HID, HID)), _full((HID, HID)),
            _full((1, HID)),
        ],
        out_specs=pl.BlockSpec((B_B, HID), lambda i: (i, 0)),
        out_shape=jax.ShapeDtypeStruct((N, HID), jnp.float32),
        compiler_params=pltpu.CompilerParams(
            vmem_limit_bytes=100 * 1024 * 1024),
    )(exbf, g, fh, f, wmpT, bmp2, w1, w2, w3, bl2)
    return out
